# baseline (speedup 1.0000x reference)
import functools

import jax
import jax.numpy as jnp
from jax import lax
from jax.experimental import pallas as pl
from jax.experimental.pallas import tpu as pltpu
from jax.experimental.pallas import tpu_sc as plsc

NC = 2
NS = 16
NW = NC * NS
SUB = 2
NSEG = NW * SUB
CHUNK = 32
BATCH = 1024
CPB = BATCH // CHUNK


def _sc_aggregate(x, ecol, erow, eval_, starts, n_pad, rps):
    d = x.shape[1]
    nk = d // 16
    mesh = plsc.VectorSubcoreMesh(core_axis_name="c", subcore_axis_name="s")

    @functools.partial(
        pl.kernel,
        out_type=jax.ShapeDtypeStruct((n_pad, d), jnp.float32),
        mesh=mesh,
        scratch_types=[
            pltpu.VMEM((80,), jnp.int32),
            pltpu.VMEM((BATCH,), jnp.int32),
            pltpu.VMEM((BATCH + 16,), jnp.int32),
            pltpu.VMEM((BATCH + 16,), jnp.float32),
            pltpu.VMEM((2, CHUNK, d), jnp.float32),
            pltpu.VMEM((rps + 1, d), jnp.float32),
            pltpu.SemaphoreType.DMA,
            pltpu.SemaphoreType.DMA,
        ],
    )
    def agg_kernel(x_hbm, ecol_hbm, erow_hbm, eval_hbm, starts_hbm, out_hbm,
                   st_v, colb, rowb, valb, gbuf, acc, sem0, sem1):
        sems = (sem0, sem1)
        wid = lax.axis_index("s") * NC + lax.axis_index("c")
        pltpu.sync_copy(starts_hbm, st_v)
        zeros16 = jnp.zeros((16,), jnp.float32)

        for sub in range(SUB):
            seg = wid * SUB + sub
            row_base = seg * rps
            sv = st_v[pl.ds(seg, 16)]
            e_lo = sv[0]
            e_hi = sv[1]
            a0 = (e_lo // CHUNK) * CHUNK
            nch = (e_hi - a0 + CHUNK - 1) // CHUNK
            nb = (nch + CPB - 1) // CPB

            def zero_row(r, _):
                for k in range(nk):
                    acc[r, pl.ds(k * 16, 16)] = zeros16
                return _
            lax.fori_loop(0, rps + 1, zero_row, None)

            def start_gather(c, par):
                idx = colb.at[pl.ds(c * CHUNK, CHUNK)]
                return pltpu.async_copy(x_hbm.at[idx], gbuf.at[par], sems[par])

            def batch_body(b, _):
                eb = a0 + b * BATCH
                pltpu.sync_copy(ecol_hbm.at[pl.ds(eb, BATCH)], colb)
                pltpu.sync_copy(erow_hbm.at[pl.ds(eb, BATCH)],
                                rowb.at[pl.ds(0, BATCH)])
                pltpu.sync_copy(eval_hbm.at[pl.ds(eb, BATCH)],
                                valb.at[pl.ds(0, BATCH)])
                ncb = jnp.minimum(CPB, nch - b * CPB)

                @pl.when(ncb > 0)
                def _():
                    start_gather(0, 0)

                @pl.when(ncb > 1)
                def _():
                    start_gather(1, 1)

                def process_chunk(c, par):
                    pltpu.make_async_copy(
                        x_hbm.at[colb.at[pl.ds(c * CHUNK, CHUNK)]],
                        gbuf.at[par], sems[par]).wait()

                    @plsc.parallel_loop(0, CHUNK, unroll=4)
                    def edge_body(j):
                        e = c * CHUNK + j
                        rv = rowb[pl.ds(e, 16)]
                        vva = valb[pl.ds(e, 16)]
                        rl = rv[0] - row_base
                        ok = (rl >= 0) & (rl < rps)
                        rl = jnp.where(ok, rl, rps)
                        vv = jnp.full((16,), vva[0], jnp.float32)
                        for k in range(nk):
                            g = gbuf[par, j, pl.ds(k * 16, 16)]
                            plsc.addupdate(acc.at[rl, pl.ds(k * 16, 16)],
                                           vv * g)

                    @pl.when(c + 2 < ncb)
                    def _():
                        start_gather(c + 2, par)

                def pair_body(i, _):
                    for par in range(2):
                        c = 2 * i + par

                        @pl.when(c < ncb)
                        def _():
                            process_chunk(c, par)
                    return _
                lax.fori_loop(0, (ncb + 1) // 2, pair_body, None)
                return _

            lax.fori_loop(0, nb, batch_body, None)
            pltpu.sync_copy(acc.at[pl.ds(0, rps)],
                            out_hbm.at[pl.ds(row_base, rps), :])

    return agg_kernel(x, ecol, erow, eval_, starts)


def _tc_matmul(agg, w, n_out):
    m_pad, d_in = agg.shape
    d_out = w.shape[1]
    bm = 512
    grid = (m_pad // bm,)

    def mm_body(a_ref, w_ref, o_ref):
        o_ref[...] = jnp.dot(a_ref[...], w_ref[...],
                             preferred_element_type=jnp.float32)

    return pl.pallas_call(
        mm_body,
        grid=grid,
        in_specs=[
            pl.BlockSpec((bm, d_in), lambda i: (i, 0)),
            pl.BlockSpec((d_in, d_out), lambda i: (0, 0)),
        ],
        out_specs=pl.BlockSpec((bm, d_out), lambda i: (i, 0)),
        out_shape=jax.ShapeDtypeStruct((n_out, d_out), jnp.float32),
    )(agg, w)


def kernel(x, edge_row, edge_col, edge_val, kernel):
    n, d = x.shape
    e = edge_row.shape[0]
    rps = -(-n // NSEG)
    rps = -(-rps // 8) * 8
    n_pad = NSEG * rps

    bounds = jnp.arange(NSEG + 1, dtype=jnp.int32) * rps
    starts = jnp.searchsorted(edge_row, bounds).astype(jnp.int32)
    starts = jnp.pad(starts, (0, 80 - (NSEG + 1)), constant_values=e)

    e_pad = (-(-e // CHUNK)) * CHUNK + BATCH
    pad = e_pad - e
    ecol = jnp.pad(edge_col, (0, pad))
    erow = jnp.pad(edge_row, (0, pad), constant_values=n_pad)
    eval_ = jnp.pad(edge_val, (0, pad))

    agg = _sc_aggregate(x, ecol, erow, eval_, starts, n_pad, rps)
    return _tc_matmul(agg, kernel, n)


# speedup vs baseline: 1.2910x; 1.2910x over previous
import functools

import jax
import jax.numpy as jnp
from jax import lax
from jax.experimental import pallas as pl
from jax.experimental.pallas import tpu as pltpu
from jax.experimental.pallas import tpu_sc as plsc

NC = 2
NS = 16
NW = NC * NS
SUB = 2
NSEG = NW * SUB
CHUNK = 32
BATCH = 1024
CPB = BATCH // CHUNK


def _sc_aggregate(x, ecol, erow, eval_, starts, n_pad, rps):
    d = x.shape[1]
    nk = d // 16
    mesh = plsc.VectorSubcoreMesh(core_axis_name="c", subcore_axis_name="s")

    @functools.partial(
        pl.kernel,
        out_type=jax.ShapeDtypeStruct((n_pad, d), jnp.float32),
        mesh=mesh,
        scratch_types=[
            pltpu.VMEM((80,), jnp.int32),
            pltpu.VMEM((BATCH,), jnp.int32),
            pltpu.VMEM((BATCH + 16,), jnp.int32),
            pltpu.VMEM((BATCH + 16,), jnp.float32),
            pltpu.VMEM((2, CHUNK, d), jnp.float32),
            pltpu.VMEM((rps + 1, d), jnp.float32),
            pltpu.SemaphoreType.DMA,
            pltpu.SemaphoreType.DMA,
        ],
    )
    def agg_kernel(x_hbm, ecol_hbm, erow_hbm, eval_hbm, starts_hbm, out_hbm,
                   st_v, colb, rowb, valb, gbuf, acc, sem0, sem1):
        sems = (sem0, sem1)
        wid = lax.axis_index("s") * NC + lax.axis_index("c")
        pltpu.sync_copy(starts_hbm, st_v)
        zeros16 = jnp.zeros((16,), jnp.float32)

        for sub in range(SUB):
            seg = wid * SUB + sub
            row_base = seg * rps
            sv = st_v[pl.ds(seg, 16)]
            e_lo = sv[0]
            e_hi = sv[1]
            a0 = (e_lo // CHUNK) * CHUNK
            nch = (e_hi - a0 + CHUNK - 1) // CHUNK
            nb = (nch + CPB - 1) // CPB

            def zero_row(r, _):
                for k in range(nk):
                    acc[r, pl.ds(k * 16, 16)] = zeros16
                return _
            lax.fori_loop(0, rps + 1, zero_row, None)

            def start_gather(c, par):
                idx = colb.at[pl.ds(c * CHUNK, CHUNK)]
                return pltpu.async_copy(x_hbm.at[idx], gbuf.at[par], sems[par])

            def batch_body(b, _):
                eb = a0 + b * BATCH
                pltpu.sync_copy(ecol_hbm.at[pl.ds(eb, BATCH)], colb)
                pltpu.sync_copy(erow_hbm.at[pl.ds(eb, BATCH)],
                                rowb.at[pl.ds(0, BATCH)])
                pltpu.sync_copy(eval_hbm.at[pl.ds(eb, BATCH)],
                                valb.at[pl.ds(0, BATCH)])
                ncb = jnp.minimum(CPB, nch - b * CPB)

                @pl.when(ncb > 0)
                def _():
                    start_gather(0, 0)

                @pl.when(ncb > 1)
                def _():
                    start_gather(1, 1)

                def process_chunk(c, par):
                    pltpu.make_async_copy(
                        x_hbm.at[colb.at[pl.ds(c * CHUNK, CHUNK)]],
                        gbuf.at[par], sems[par]).wait()

                    @plsc.parallel_loop(0, CHUNK, unroll=1)
                    def edge_body(j):
                        e = c * CHUNK + j
                        rv = rowb[pl.ds(e, 16)]
                        vva = valb[pl.ds(e, 16)]
                        rl = rv[0] - row_base
                        ok = (rl >= 0) & (rl < rps)
                        rl = jnp.where(ok, rl, rps)
                        vv = jnp.full((16,), vva[0], jnp.float32)
                        for k in range(nk):
                            g = gbuf[par, j, pl.ds(k * 16, 16)]
                            plsc.addupdate(acc.at[rl, pl.ds(k * 16, 16)],
                                           vv * g)

                    @pl.when(c + 2 < ncb)
                    def _():
                        start_gather(c + 2, par)

                def pair_body(i, _):
                    for par in range(2):
                        c = 2 * i + par

                        @pl.when(c < ncb)
                        def _():
                            process_chunk(c, par)
                    return _
                lax.fori_loop(0, (ncb + 1) // 2, pair_body, None)
                return _

            lax.fori_loop(0, nb, batch_body, None)
            pltpu.sync_copy(acc.at[pl.ds(0, rps)],
                            out_hbm.at[pl.ds(row_base, rps), :])

    return agg_kernel(x, ecol, erow, eval_, starts)


def _tc_matmul(agg, w, n_out):
    m_pad, d_in = agg.shape
    d_out = w.shape[1]
    bm = 512
    grid = (m_pad // bm,)

    def mm_body(a_ref, w_ref, o_ref):
        o_ref[...] = jnp.dot(a_ref[...], w_ref[...],
                             preferred_element_type=jnp.float32)

    return pl.pallas_call(
        mm_body,
        grid=grid,
        in_specs=[
            pl.BlockSpec((bm, d_in), lambda i: (i, 0)),
            pl.BlockSpec((d_in, d_out), lambda i: (0, 0)),
        ],
        out_specs=pl.BlockSpec((bm, d_out), lambda i: (i, 0)),
        out_shape=jax.ShapeDtypeStruct((n_out, d_out), jnp.float32),
    )(agg, w)


def kernel(x, edge_row, edge_col, edge_val, kernel):
    n, d = x.shape
    e = edge_row.shape[0]
    rps = -(-n // NSEG)
    rps = -(-rps // 8) * 8
    n_pad = NSEG * rps

    bounds = jnp.arange(NSEG + 1, dtype=jnp.int32) * rps
    starts = jnp.searchsorted(edge_row, bounds).astype(jnp.int32)
    starts = jnp.pad(starts, (0, 80 - (NSEG + 1)), constant_values=e)

    e_pad = (-(-e // CHUNK)) * CHUNK + BATCH
    pad = e_pad - e
    ecol = jnp.pad(edge_col, (0, pad))
    erow = jnp.pad(edge_row, (0, pad), constant_values=n_pad)
    eval_ = jnp.pad(edge_val, (0, pad))

    agg = _sc_aggregate(x, ecol, erow, eval_, starts, n_pad, rps)
    return _tc_matmul(agg, kernel, n)


# speedup vs baseline: 1.2987x; 1.0060x over previous
import functools

import jax
import jax.numpy as jnp
from jax import lax
from jax.experimental import pallas as pl
from jax.experimental.pallas import tpu as pltpu
from jax.experimental.pallas import tpu_sc as plsc

NC = 2
NS = 16
NW = NC * NS
SUB = 2
NSEG = NW * SUB
CHUNK = 32
BATCH = 1024
CPB = BATCH // CHUNK


def _sc_aggregate(x, ecol, erow, eval_, starts, n_pad, rps):
    d = x.shape[1]
    nk = d // 16
    mesh = plsc.VectorSubcoreMesh(core_axis_name="c", subcore_axis_name="s")

    @functools.partial(
        pl.kernel,
        out_type=jax.ShapeDtypeStruct((n_pad, d), jnp.float32),
        mesh=mesh,
        scratch_types=[
            pltpu.VMEM((80,), jnp.int32),
            pltpu.VMEM((BATCH,), jnp.int32),
            pltpu.VMEM((BATCH + 16,), jnp.int32),
            pltpu.VMEM((BATCH + 16,), jnp.float32),
            pltpu.VMEM((2, CHUNK, d), jnp.float32),
            pltpu.VMEM((rps + 1, d), jnp.float32),
            pltpu.SemaphoreType.DMA,
            pltpu.SemaphoreType.DMA,
        ],
    )
    def agg_kernel(x_hbm, ecol_hbm, erow_hbm, eval_hbm, starts_hbm, out_hbm,
                   st_v, colb, rowb, valb, gbuf, acc, sem0, sem1):
        sems = (sem0, sem1)
        wid = lax.axis_index("s") * NC + lax.axis_index("c")
        pltpu.sync_copy(starts_hbm, st_v)
        zeros16 = jnp.zeros((16,), jnp.float32)

        for sub in range(SUB):
            seg = wid * SUB + sub
            row_base = seg * rps
            sv = st_v[pl.ds(seg, 16)]
            e_lo = sv[0]
            e_hi = sv[1]
            a0 = (e_lo // CHUNK) * CHUNK
            nch = (e_hi - a0 + CHUNK - 1) // CHUNK
            nb = (nch + CPB - 1) // CPB

            def zero_row(r, _):
                for k in range(nk):
                    acc[r, pl.ds(k * 16, 16)] = zeros16
                return _
            lax.fori_loop(0, rps + 1, zero_row, None)

            def start_gather(c, par):
                idx = colb.at[pl.ds(c * CHUNK, CHUNK)]
                return pltpu.async_copy(x_hbm.at[idx], gbuf.at[par], sems[par])

            def batch_body(b, _):
                eb = a0 + b * BATCH
                pltpu.sync_copy(ecol_hbm.at[pl.ds(eb, BATCH)], colb)
                pltpu.sync_copy(erow_hbm.at[pl.ds(eb, BATCH)],
                                rowb.at[pl.ds(0, BATCH)])
                pltpu.sync_copy(eval_hbm.at[pl.ds(eb, BATCH)],
                                valb.at[pl.ds(0, BATCH)])
                ncb = jnp.minimum(CPB, nch - b * CPB)

                @pl.when(ncb > 0)
                def _():
                    start_gather(0, 0)

                @pl.when(ncb > 1)
                def _():
                    start_gather(1, 1)

                def process_chunk(c, par):
                    pltpu.make_async_copy(
                        x_hbm.at[colb.at[pl.ds(c * CHUNK, CHUNK)]],
                        gbuf.at[par], sems[par]).wait()

                    @plsc.parallel_loop(0, CHUNK, unroll=2)
                    def edge_body(j):
                        e = c * CHUNK + j
                        rv = rowb[pl.ds(e, 16)]
                        vva = valb[pl.ds(e, 16)]
                        rl = rv[0] - row_base
                        ok = (rl >= 0) & (rl < rps)
                        rl = jnp.where(ok, rl, rps)
                        vv = jnp.full((16,), vva[0], jnp.float32)
                        for k in range(nk):
                            g = gbuf[par, j, pl.ds(k * 16, 16)]
                            plsc.addupdate(acc.at[rl, pl.ds(k * 16, 16)],
                                           vv * g)

                    @pl.when(c + 2 < ncb)
                    def _():
                        start_gather(c + 2, par)

                def pair_body(i, _):
                    for par in range(2):
                        c = 2 * i + par

                        @pl.when(c < ncb)
                        def _():
                            process_chunk(c, par)
                    return _
                lax.fori_loop(0, (ncb + 1) // 2, pair_body, None)
                return _

            lax.fori_loop(0, nb, batch_body, None)
            pltpu.sync_copy(acc.at[pl.ds(0, rps)],
                            out_hbm.at[pl.ds(row_base, rps), :])

    return agg_kernel(x, ecol, erow, eval_, starts)


def _tc_matmul(agg, w, n_out):
    m_pad, d_in = agg.shape
    d_out = w.shape[1]
    bm = 512
    grid = (m_pad // bm,)

    def mm_body(a_ref, w_ref, o_ref):
        o_ref[...] = jnp.dot(a_ref[...], w_ref[...],
                             preferred_element_type=jnp.float32)

    return pl.pallas_call(
        mm_body,
        grid=grid,
        in_specs=[
            pl.BlockSpec((bm, d_in), lambda i: (i, 0)),
            pl.BlockSpec((d_in, d_out), lambda i: (0, 0)),
        ],
        out_specs=pl.BlockSpec((bm, d_out), lambda i: (i, 0)),
        out_shape=jax.ShapeDtypeStruct((n_out, d_out), jnp.float32),
    )(agg, w)


def kernel(x, edge_row, edge_col, edge_val, kernel):
    n, d = x.shape
    e = edge_row.shape[0]
    rps = -(-n // NSEG)
    rps = -(-rps // 8) * 8
    n_pad = NSEG * rps

    bounds = jnp.arange(NSEG + 1, dtype=jnp.int32) * rps
    starts = jnp.searchsorted(edge_row, bounds).astype(jnp.int32)
    starts = jnp.pad(starts, (0, 80 - (NSEG + 1)), constant_values=e)

    e_pad = (-(-e // CHUNK)) * CHUNK + BATCH
    pad = e_pad - e
    ecol = jnp.pad(edge_col, (0, pad))
    erow = jnp.pad(edge_row, (0, pad), constant_values=n_pad)
    eval_ = jnp.pad(edge_val, (0, pad))

    agg = _sc_aggregate(x, ecol, erow, eval_, starts, n_pad, rps)
    return _tc_matmul(agg, kernel, n)


# speedup vs baseline: 1.5445x; 1.1892x over previous
import functools

import jax
import jax.numpy as jnp
from jax import lax
from jax.experimental import pallas as pl
from jax.experimental.pallas import tpu as pltpu
from jax.experimental.pallas import tpu_sc as plsc

NC = 2
NS = 16
NW = NC * NS
SUB = 2
NSEG = NW * SUB
CHUNK = 32
BATCH = 1024
CPB = BATCH // CHUNK


def _sc_aggregate(x, ecol, erow, eval_, starts, zrows, n_pad, rps):
    d = x.shape[1]
    nk = d // 16
    mesh = plsc.VectorSubcoreMesh(core_axis_name="c", subcore_axis_name="s")

    @functools.partial(
        pl.kernel,
        out_type=jax.ShapeDtypeStruct((n_pad, d), jnp.float32),
        mesh=mesh,
        scratch_types=[
            pltpu.VMEM((80,), jnp.int32),
            pltpu.VMEM((BATCH,), jnp.int32),
            pltpu.VMEM((BATCH + 16,), jnp.int32),
            pltpu.VMEM((BATCH + 16,), jnp.float32),
            pltpu.VMEM((2, CHUNK, d), jnp.float32),
            pltpu.VMEM((rps + 1, d), jnp.float32),
            pltpu.SemaphoreType.DMA,
            pltpu.SemaphoreType.DMA,
            pltpu.SemaphoreType.DMA,
        ],
    )
    def agg_kernel(x_hbm, ecol_hbm, erow_hbm, eval_hbm, starts_hbm, z_hbm,
                   out_hbm, st_v, colb, rowb, valb, gbuf, acc,
                   sem0, sem1, sem2):
        sems = (sem0, sem1)
        wid = lax.axis_index("s") * NC + lax.axis_index("c")
        pltpu.sync_copy(starts_hbm, st_v)

        for sub in range(SUB):
            seg = wid * SUB + sub
            row_base = seg * rps
            sv = st_v[pl.ds(seg, 16)]
            e_lo = sv[0]
            e_hi = sv[1]
            a0 = (e_lo // CHUNK) * CHUNK
            nch = (e_hi - a0 + CHUNK - 1) // CHUNK
            nb = (nch + CPB - 1) // CPB

            pltpu.sync_copy(z_hbm, acc)

            def start_gather(c, par):
                idx = colb.at[pl.ds(c * CHUNK, CHUNK)]
                return pltpu.async_copy(x_hbm.at[idx], gbuf.at[par], sems[par])

            def batch_body(b, _):
                eb = a0 + b * BATCH
                c1 = pltpu.async_copy(ecol_hbm.at[pl.ds(eb, BATCH)],
                                      colb, sem2)
                c2 = pltpu.async_copy(erow_hbm.at[pl.ds(eb, BATCH)],
                                      rowb.at[pl.ds(0, BATCH)], sem2)
                c3 = pltpu.async_copy(eval_hbm.at[pl.ds(eb, BATCH)],
                                      valb.at[pl.ds(0, BATCH)], sem2)
                c1.wait()
                c2.wait()
                c3.wait()
                ncb = jnp.minimum(CPB, nch - b * CPB)

                @pl.when(ncb > 0)
                def _():
                    start_gather(0, 0)

                @pl.when(ncb > 1)
                def _():
                    start_gather(1, 1)

                def process_chunk(c, par):
                    pltpu.make_async_copy(
                        x_hbm.at[colb.at[pl.ds(c * CHUNK, CHUNK)]],
                        gbuf.at[par], sems[par]).wait()

                    @plsc.parallel_loop(0, CHUNK, unroll=2)
                    def edge_body(j):
                        e = c * CHUNK + j
                        rv = rowb[pl.ds(e, 16)]
                        vva = valb[pl.ds(e, 16)]
                        rl = rv[0] - row_base
                        ok = (rl >= 0) & (rl < rps)
                        rl = jnp.where(ok, rl, rps)
                        vv = jnp.full((16,), vva[0], jnp.float32)
                        for k in range(nk):
                            g = gbuf[par, j, pl.ds(k * 16, 16)]
                            plsc.addupdate(acc.at[rl, pl.ds(k * 16, 16)],
                                           vv * g)

                    @pl.when(c + 2 < ncb)
                    def _():
                        start_gather(c + 2, par)

                def pair_body(i, _):
                    for par in range(2):
                        c = 2 * i + par

                        @pl.when(c < ncb)
                        def _():
                            process_chunk(c, par)
                    return _
                lax.fori_loop(0, (ncb + 1) // 2, pair_body, None)
                return _

            lax.fori_loop(0, nb, batch_body, None)
            pltpu.sync_copy(acc.at[pl.ds(0, rps)],
                            out_hbm.at[pl.ds(row_base, rps), :])

    return agg_kernel(x, ecol, erow, eval_, starts, zrows)


def _tc_matmul(agg, w, n_out):
    m_pad, d_in = agg.shape
    d_out = w.shape[1]
    bm = 512
    grid = (m_pad // bm,)

    def mm_body(a_ref, w_ref, o_ref):
        o_ref[...] = jnp.dot(a_ref[...], w_ref[...],
                             preferred_element_type=jnp.float32)

    return pl.pallas_call(
        mm_body,
        grid=grid,
        in_specs=[
            pl.BlockSpec((bm, d_in), lambda i: (i, 0)),
            pl.BlockSpec((d_in, d_out), lambda i: (0, 0)),
        ],
        out_specs=pl.BlockSpec((bm, d_out), lambda i: (i, 0)),
        out_shape=jax.ShapeDtypeStruct((n_out, d_out), jnp.float32),
    )(agg, w)


def kernel(x, edge_row, edge_col, edge_val, kernel):
    n, d = x.shape
    e = edge_row.shape[0]
    rps = -(-n // NSEG)
    rps = -(-rps // 8) * 8
    n_pad = NSEG * rps

    bounds = jnp.arange(NSEG + 1, dtype=jnp.int32) * rps
    starts = jnp.searchsorted(edge_row, bounds,
                              method="compare_all").astype(jnp.int32)
    starts = jnp.pad(starts, (0, 80 - (NSEG + 1)), constant_values=e)

    e_pad = (-(-e // CHUNK)) * CHUNK + BATCH
    pad = e_pad - e
    ecol = jnp.pad(edge_col, (0, pad))
    erow = jnp.pad(edge_row, (0, pad), constant_values=n_pad)
    eval_ = jnp.pad(edge_val, (0, pad))

    zrows = jnp.zeros((rps + 1, d), jnp.float32)
    agg = _sc_aggregate(x, ecol, erow, eval_, starts, zrows, n_pad, rps)
    return _tc_matmul(agg, kernel, n)
